# baseline (speedup 1.0000x reference)
import functools

import numpy as np

import jax
import jax.numpy as jnp
from jax import lax
from jax.experimental import pallas as pl
from jax.experimental.pallas import tpu as pltpu
from jax.experimental.pallas import tpu_sc as plsc

BATCH = 16384
D = 128
LANES = 16
GROUPS = D // (2 * LANES)

TC_ROWS = 8192
TC_BLK = 1024
NCLS_PAD = 1024

_info = plsc.get_sparse_core_info()
NC, NS = _info.num_cores, _info.num_subcores
NW = NC * NS
SC_ROWS = BATCH - TC_ROWS
ROWS_W = SC_ROWS // NW
CHUNK = 128
CHUNKS = (128, 128)
OFFS = (0, 128)
NCHUNK = len(CHUNKS)
assert sum(CHUNKS) == ROWS_W


def _make_sc_call():
    mesh = plsc.VectorSubcoreMesh(core_axis_name="c", subcore_axis_name="s")

    @functools.partial(
        pl.kernel,
        mesh=mesh,
        out_type=jax.ShapeDtypeStruct((NW, LANES), jnp.float32),
        scratch_types=[
            pltpu.VMEM((ROWS_W,), jnp.int32),
            pltpu.VMEM((2, CHUNK, D), jnp.float32),
            pltpu.VMEM((NCHUNK, CHUNK, D), jnp.float32),
            pltpu.VMEM((LANES,), jnp.float32),
            pltpu.SemaphoreType.DMA,
            pltpu.SemaphoreType.DMA,
            pltpu.SemaphoreType.DMA,
            pltpu.SemaphoreType.DMA,
            pltpu.SemaphoreType.DMA,
        ],
    )
    def sc_center_loss(feat_hbm, label_hbm, ctable_hbm, out_hbm,
                       idx_v, cent_v, feat_v, out_v,
                       gsem0, gsem1, fsem0, fsem1, fsem2):
        wid = lax.axis_index("s") * NC + lax.axis_index("c")
        base = TC_ROWS + wid * ROWS_W
        gsems = (gsem0, gsem1)
        fsems = (fsem0, fsem1, fsem2)

        fcopies = [
            pltpu.async_copy(
                feat_hbm.at[pl.ds(base + OFFS[c], CHUNKS[c])],
                feat_v.at[c].at[pl.ds(0, CHUNKS[c])], fsems[c])
            for c in range(NCHUNK)
        ]
        pltpu.sync_copy(label_hbm.at[pl.ds(base, ROWS_W)], idx_v)

        def start(c, slot):
            g = pltpu.async_copy(
                ctable_hbm.at[idx_v.at[pl.ds(OFFS[c], CHUNKS[c])]],
                cent_v.at[slot].at[pl.ds(0, CHUNKS[c])], gsems[slot])
            return g, fcopies[c]

        def compute(c, slot, accs):
            fv = feat_v.at[c]
            cv = cent_v.at[slot]

            def body(i, accs):
                new = list(accs)
                for g in range(GROUPS):
                    w = jax.lax.bitcast_convert_type(
                        cv[i, pl.ds(g * LANES, LANES)], jnp.int32)
                    clo = jax.lax.bitcast_convert_type(w << 16, jnp.float32)
                    chi = jax.lax.bitcast_convert_type(w & jnp.int32(-65536),
                                                       jnp.float32)
                    flo = fv[i, pl.ds(g * LANES, LANES)]
                    fhi = fv[i, pl.ds(D // 2 + g * LANES, LANES)]
                    dlo = flo - clo
                    dhi = fhi - chi
                    new[2 * g] = new[2 * g] + dlo * dlo
                    new[2 * g + 1] = new[2 * g + 1] + dhi * dhi
                return tuple(new)

            return lax.fori_loop(0, CHUNKS[c], body, accs)

        accs = tuple(jnp.zeros((LANES,), jnp.float32) for _ in range(2 * GROUPS))
        copies = {0: start(0, 0)}
        for c in range(NCHUNK):
            if c + 1 < NCHUNK:
                copies[c + 1] = start(c + 1, (c + 1) % 2)
            g, f = copies.pop(c)
            g.wait()
            f.wait()
            accs = compute(c, c % 2, accs)

        total = accs[0]
        for j in range(1, 2 * GROUPS):
            total = total + accs[j]
        out_v[...] = total * 0.5
        pltpu.sync_copy(out_v, out_hbm.at[wid])

    return sc_center_loss


_sc_center_loss = _make_sc_call()


def _pack_centers(centers):
    bits = jax.lax.bitcast_convert_type(centers, jnp.int32)
    rne = (bits + jnp.int32(0x7FFF) + ((bits >> 16) & 1)) >> 16
    rne = rne & jnp.int32(0xFFFF)
    words = rne[:, :D // 2] | (rne[:, D // 2:] << 16)
    packed = jnp.pad(words, ((0, 0), (0, D // 2)))
    return jax.lax.bitcast_convert_type(packed, jnp.float32)


def _tc_body(lab_ref, feat_ref, cent_ref, out_ref):
    k = pl.program_id(0)
    labs = jnp.broadcast_to(lab_ref[0], (NCLS_PAD, TC_BLK))
    iot = jax.lax.broadcasted_iota(jnp.int32, (NCLS_PAD, TC_BLK), 0)
    onehot_t = jnp.where(labs == iot, 1.0, 0.0).astype(jnp.bfloat16)
    gathered = jax.lax.dot_general(
        onehot_t, cent_ref[...], (((0,), (0,)), ((), ())),
        preferred_element_type=jnp.float32)
    dmat = feat_ref[...] - gathered
    part = jnp.sum(dmat * dmat, axis=0, keepdims=True)

    @pl.when(k == 0)
    def _init():
        out_ref[...] = jnp.zeros_like(out_ref)

    out_ref[...] += part


def _make_tc_call():
    return pl.pallas_call(
        _tc_body,
        grid=(TC_ROWS // TC_BLK,),
        in_specs=[
            pl.BlockSpec((1, 1, TC_BLK), lambda k: (k, 0, 0)),
            pl.BlockSpec((TC_BLK, D), lambda k: (k, 0)),
            pl.BlockSpec((NCLS_PAD, D), lambda k: (0, 0)),
        ],
        out_specs=pl.BlockSpec((1, D), lambda k: (0, 0)),
        out_shape=jax.ShapeDtypeStruct((1, D), jnp.float32),
    )


_tc_center_loss = _make_tc_call()


def kernel(feat, label, centers):
    label32 = label.astype(jnp.int32)
    sc_part = _sc_center_loss(feat, label32, _pack_centers(centers))
    cpad = jnp.pad(centers.astype(jnp.bfloat16),
                   ((0, NCLS_PAD - centers.shape[0]), (0, 0)))
    tc_part = _tc_center_loss(
        label32.reshape(BATCH // TC_BLK, 1, TC_BLK), feat, cpad)
    return jnp.sum(sc_part) + 0.5 * jnp.sum(tc_part)


# speedup vs baseline: 1.1062x; 1.1062x over previous
import functools

import numpy as np

import jax
import jax.numpy as jnp
from jax import lax
from jax.experimental import pallas as pl
from jax.experimental.pallas import tpu as pltpu
from jax.experimental.pallas import tpu_sc as plsc

BATCH = 16384
D = 128
LANES = 16
GROUPS = D // (2 * LANES)

TC_ROWS = 4096
TC_BLK = 1024
NCLS_PAD = 1024

_info = plsc.get_sparse_core_info()
NC, NS = _info.num_cores, _info.num_subcores
NW = NC * NS
SC_ROWS = BATCH - TC_ROWS
ROWS_W = SC_ROWS // NW
CHUNK = 128
CHUNKS = (128, 128, 128)
OFFS = (0, 128, 256)
NCHUNK = len(CHUNKS)
assert sum(CHUNKS) == ROWS_W


def _make_sc_call():
    mesh = plsc.VectorSubcoreMesh(core_axis_name="c", subcore_axis_name="s")

    @functools.partial(
        pl.kernel,
        mesh=mesh,
        out_type=jax.ShapeDtypeStruct((NW, LANES), jnp.float32),
        scratch_types=[
            pltpu.VMEM((ROWS_W,), jnp.int32),
            pltpu.VMEM((2, CHUNK, D), jnp.float32),
            pltpu.VMEM((NCHUNK, CHUNK, D), jnp.float32),
            pltpu.VMEM((LANES,), jnp.float32),
            pltpu.SemaphoreType.DMA,
            pltpu.SemaphoreType.DMA,
            pltpu.SemaphoreType.DMA,
            pltpu.SemaphoreType.DMA,
            pltpu.SemaphoreType.DMA,
        ],
    )
    def sc_center_loss(feat_hbm, label_hbm, ctable_hbm, out_hbm,
                       idx_v, cent_v, feat_v, out_v,
                       gsem0, gsem1, fsem0, fsem1, fsem2):
        wid = lax.axis_index("s") * NC + lax.axis_index("c")
        base = TC_ROWS + wid * ROWS_W
        gsems = (gsem0, gsem1)
        fsems = (fsem0, fsem1, fsem2)

        fcopies = [
            pltpu.async_copy(
                feat_hbm.at[pl.ds(base + OFFS[c], CHUNKS[c])],
                feat_v.at[c].at[pl.ds(0, CHUNKS[c])], fsems[c])
            for c in range(NCHUNK)
        ]
        pltpu.sync_copy(label_hbm.at[pl.ds(base, ROWS_W)], idx_v)

        def start(c, slot):
            g = pltpu.async_copy(
                ctable_hbm.at[idx_v.at[pl.ds(OFFS[c], CHUNKS[c])]],
                cent_v.at[slot].at[pl.ds(0, CHUNKS[c])], gsems[slot])
            return g, fcopies[c]

        def compute(c, slot, accs):
            fv = feat_v.at[c]
            cv = cent_v.at[slot]

            def body(i, accs):
                new = list(accs)
                for g in range(GROUPS):
                    w = jax.lax.bitcast_convert_type(
                        cv[i, pl.ds(g * LANES, LANES)], jnp.int32)
                    clo = jax.lax.bitcast_convert_type(w << 16, jnp.float32)
                    chi = jax.lax.bitcast_convert_type(w & jnp.int32(-65536),
                                                       jnp.float32)
                    flo = fv[i, pl.ds(g * LANES, LANES)]
                    fhi = fv[i, pl.ds(D // 2 + g * LANES, LANES)]
                    dlo = flo - clo
                    dhi = fhi - chi
                    new[2 * g] = new[2 * g] + dlo * dlo
                    new[2 * g + 1] = new[2 * g + 1] + dhi * dhi
                return tuple(new)

            return lax.fori_loop(0, CHUNKS[c], body, accs)

        accs = tuple(jnp.zeros((LANES,), jnp.float32) for _ in range(2 * GROUPS))
        copies = {0: start(0, 0)}
        for c in range(NCHUNK):
            if c + 1 < NCHUNK:
                copies[c + 1] = start(c + 1, (c + 1) % 2)
            g, f = copies.pop(c)
            g.wait()
            f.wait()
            accs = compute(c, c % 2, accs)

        total = accs[0]
        for j in range(1, 2 * GROUPS):
            total = total + accs[j]
        out_v[...] = total * 0.5
        pltpu.sync_copy(out_v, out_hbm.at[wid])

    return sc_center_loss


_sc_center_loss = _make_sc_call()


def _pack_centers(centers):
    bits = jax.lax.bitcast_convert_type(centers, jnp.int32)
    rne = (bits + jnp.int32(0x7FFF) + ((bits >> 16) & 1)) >> 16
    rne = rne & jnp.int32(0xFFFF)
    words = rne[:, :D // 2] | (rne[:, D // 2:] << 16)
    packed = jnp.pad(words, ((0, 0), (0, D // 2)))
    return jax.lax.bitcast_convert_type(packed, jnp.float32)


def _tc_body(lab_ref, feat_ref, cent_ref, out_ref):
    k = pl.program_id(0)
    labs = jnp.broadcast_to(lab_ref[0], (NCLS_PAD, TC_BLK))
    iot = jax.lax.broadcasted_iota(jnp.int32, (NCLS_PAD, TC_BLK), 0)
    onehot_t = jnp.where(labs == iot, 1.0, 0.0).astype(jnp.bfloat16)
    gathered = jax.lax.dot_general(
        onehot_t, cent_ref[...], (((0,), (0,)), ((), ())),
        preferred_element_type=jnp.float32)
    dmat = feat_ref[...] - gathered
    part = jnp.sum(dmat * dmat, axis=0, keepdims=True)

    @pl.when(k == 0)
    def _init():
        out_ref[...] = jnp.zeros_like(out_ref)

    out_ref[...] += part


def _make_tc_call():
    return pl.pallas_call(
        _tc_body,
        grid=(TC_ROWS // TC_BLK,),
        in_specs=[
            pl.BlockSpec((1, 1, TC_BLK), lambda k: (k, 0, 0)),
            pl.BlockSpec((TC_BLK, D), lambda k: (k, 0)),
            pl.BlockSpec((NCLS_PAD, D), lambda k: (0, 0)),
        ],
        out_specs=pl.BlockSpec((1, D), lambda k: (0, 0)),
        out_shape=jax.ShapeDtypeStruct((1, D), jnp.float32),
    )


_tc_center_loss = _make_tc_call()


def kernel(feat, label, centers):
    label32 = label.astype(jnp.int32)
    sc_part = _sc_center_loss(feat, label32, _pack_centers(centers))
    cpad = jnp.pad(centers.astype(jnp.bfloat16),
                   ((0, NCLS_PAD - centers.shape[0]), (0, 0)))
    tc_part = _tc_center_loss(
        label32.reshape(BATCH // TC_BLK, 1, TC_BLK), feat, cpad)
    return jnp.sum(sc_part) + 0.5 * jnp.sum(tc_part)
